# baseline (speedup 1.0000x reference)
import functools

import jax
import jax.numpy as jnp
from jax import lax
from jax.experimental import pallas as pl
from jax.experimental.pallas import tpu as pltpu
from jax.experimental.pallas import tpu_sc as plsc

_N_NODES = 10000
_N_EDGES = 320000
_D = 128
_NSEG = 10000
_NC = 2
_NS = 16
_NW = _NC * _NS
_K = 128
_G = _N_EDGES // _K
_NITER = -(-_G // _NW)
_HALF = (_NITER + 1) // 2
_RPT = 624


def _sc_body(x_hbm, tn_hbm, ti_hbm, z0_hbm, out_hbm,
             idxn0, idxn1, idxs0, idxs1, rows0, rows1, zsh,
             sin0, sin1, sis0, sis1, sg0, sg1, ss0, ss1):
    c = lax.axis_index("c")
    s = lax.axis_index("s")
    wid = s * _NC + c
    idxn = (idxn0, idxn1)
    idxs = (idxs0, idxs1)
    rows = (rows0, rows1)
    sin = (sin0, sin1)
    sis = (sis0, sis1)
    sg = (sg0, sg1)
    ss = (ss0, ss1)

    r0 = s * _RPT
    _RPT_LAST = _NSEG - (_NS - 1) * _RPT

    @pl.when(s < _NS - 1)
    def _zero_main():
        pltpu.sync_copy(z0_hbm.at[pl.ds(0, _RPT)], zsh.at[pl.ds(r0, _RPT)])

    @pl.when(s == _NS - 1)
    def _zero_last():
        pltpu.sync_copy(z0_hbm.at[pl.ds(0, _RPT_LAST)],
                        zsh.at[pl.ds(r0, _RPT_LAST)])

    plsc.subcore_barrier()

    def step(t, carry):
        for b in range(2):
            cc = 2 * t + b
            g = wid + _NW * cc

            @pl.when(g < _G)
            def _start(b=b, cc=cc, g=g):
                e0 = g * _K

                @pl.when(cc >= 2)
                def _drain_prev():
                    pltpu.make_async_copy(rows[b], zsh.at[idxs[b]], ss[b]).wait()

                pltpu.async_copy(ti_hbm.at[pl.ds(e0, _K)], idxs[b], sis[b])

                @pl.when(cc < 2)
                def _first_idxn():
                    pltpu.async_copy(tn_hbm.at[pl.ds(e0, _K)], idxn[b], sin[b])

                pltpu.make_async_copy(tn_hbm.at[pl.ds(e0, _K)], idxn[b],
                                      sin[b]).wait()
                pltpu.async_copy(x_hbm.at[idxn[b]], rows[b], sg[b])

        for b in range(2):
            cc = 2 * t + b
            g = wid + _NW * cc

            @pl.when(g < _G)
            def _finish(b=b, cc=cc, g=g):
                pltpu.make_async_copy(x_hbm.at[idxn[b]], rows[b], sg[b]).wait()
                pltpu.make_async_copy(ti_hbm.at[pl.ds(g * _K, _K)], idxs[b],
                                      sis[b]).wait()
                pltpu.async_copy(rows[b], zsh.at[idxs[b]], ss[b], add=True)

                g2 = g + 2 * _NW

                @pl.when(g2 < _G)
                def _prefetch_idxn():
                    pltpu.async_copy(tn_hbm.at[pl.ds(g2 * _K, _K)], idxn[b],
                                     sin[b])

        return carry

    lax.fori_loop(0, _HALF, step, 0)

    for b in range(2):
        pltpu.make_async_copy(rows[b], zsh.at[idxs[b]], ss[b]).wait()
    plsc.subcore_barrier()

    @pl.when(s < _NS - 1)
    def _out_main():
        pltpu.sync_copy(zsh.at[pl.ds(r0, _RPT)],
                        out_hbm.at[pl.ds(c * _NSEG + r0, _RPT)])

    @pl.when(s == _NS - 1)
    def _out_last():
        pltpu.sync_copy(zsh.at[pl.ds(r0, _RPT_LAST)],
                        out_hbm.at[pl.ds(c * _NSEG + r0, _RPT_LAST)])


_sc_segment_sum = functools.partial(
    pl.kernel,
    out_type=jax.ShapeDtypeStruct((_NC * _NSEG, _D), jnp.float32),
    mesh=plsc.VectorSubcoreMesh(core_axis_name="c", subcore_axis_name="s",
                                num_cores=_NC, num_subcores=_NS),
    scratch_types=[
        pltpu.VMEM((_K,), jnp.int32),
        pltpu.VMEM((_K,), jnp.int32),
        pltpu.VMEM((_K,), jnp.int32),
        pltpu.VMEM((_K,), jnp.int32),
        pltpu.VMEM((_K, _D), jnp.float32),
        pltpu.VMEM((_K, _D), jnp.float32),
        pltpu.VMEM_SHARED((_NSEG, _D), jnp.float32),
        pltpu.SemaphoreType.DMA,
        pltpu.SemaphoreType.DMA,
        pltpu.SemaphoreType.DMA,
        pltpu.SemaphoreType.DMA,
        pltpu.SemaphoreType.DMA,
        pltpu.SemaphoreType.DMA,
        pltpu.SemaphoreType.DMA,
        pltpu.SemaphoreType.DMA,
    ],
)(_sc_body)


def _mlp_body(zp_ref, w1_ref, b1_ref, w2t_ref, b2_ref, o_ref):
    z = zp_ref[0:_NSEG, :] + zp_ref[_NSEG:2 * _NSEG, :]
    h = jnp.dot(z, w1_ref[...], preferred_element_type=jnp.float32)
    h = jnp.maximum(h + b1_ref[...], 0.0)
    logit = jnp.sum(h * w2t_ref[...], axis=1, keepdims=True) + b2_ref[...]
    o_ref[...] = jax.nn.sigmoid(logit)


_mlp = pl.pallas_call(
    _mlp_body,
    out_shape=jax.ShapeDtypeStruct((_NSEG, 1), jnp.float32),
)


def kernel(x, target_nodes, target_ids, W1, b1, W2, b2):
    tn = target_nodes.astype(jnp.int32)
    ti = target_ids.astype(jnp.int32)
    zeros = jnp.zeros((_NSEG - (_NS - 1) * _RPT, _D), jnp.float32)
    zparts = _sc_segment_sum(x, tn, ti, zeros)
    out = _mlp(zparts, W1, b1.reshape(1, _D), W2.reshape(1, _D),
               b2.reshape(1, 1))
    return out.reshape(_NSEG)


# speedup vs baseline: 1.0651x; 1.0651x over previous
import functools

import jax
import jax.numpy as jnp
from jax import lax
from jax.experimental import pallas as pl
from jax.experimental.pallas import tpu as pltpu
from jax.experimental.pallas import tpu_sc as plsc

_N_NODES = 10000
_N_EDGES = 320000
_D = 128
_NSEG = 10000
_NC = 2
_NS = 16
_NW = _NC * _NS
_K = 128
_G = _N_EDGES // _K
_GPAD = 8 * (-(-_G // 8))
_NBASE = _G // _NW
_NREM = _G - _NBASE * _NW
_IDXROWS = 88
_SROWS = 48
_PH1 = 40
_RPT = 624


def _sc_body(x_hbm, tn_hbm, ti_hbm, z0_hbm, out_hbm,
             idxn_all, idxs_all, rows0, rows1,
             zsh, sia, sib, sg0, sg1, ss0, ss1):
    c = lax.axis_index("c")
    s = lax.axis_index("s")
    wid = s * _NC + c
    rows = (rows0, rows1)
    sg = (sg0, sg1)
    ss = (ss0, ss1)

    S = _NBASE * wid + jnp.minimum(wid, _NREM)
    n = jnp.where(wid < _NREM, _NBASE + 1, _NBASE)
    A = (S // 8) * 8
    off = S - A

    cia = pltpu.async_copy(tn_hbm.at[pl.ds(A, _IDXROWS)], idxn_all, sia)
    cib = pltpu.async_copy(ti_hbm.at[pl.ds(A, _SROWS)], idxs_all, sib)

    r0 = s * _RPT
    _RPT_LAST = _NSEG - (_NS - 1) * _RPT

    @pl.when(s < _NS - 1)
    def _zero_main():
        pltpu.sync_copy(z0_hbm.at[pl.ds(0, _RPT)], zsh.at[pl.ds(r0, _RPT)])

    @pl.when(s == _NS - 1)
    def _zero_last():
        pltpu.sync_copy(z0_hbm.at[pl.ds(0, _RPT_LAST)],
                        zsh.at[pl.ds(r0, _RPT_LAST)])

    cia.wait()
    cib.wait()
    plsc.subcore_barrier()

    def chunk_body(b, cc, srow, guarded_drain):
        @pl.when(guarded_drain)
        def _drain_prev():
            pltpu.make_async_copy(rows[b], zsh.at[idxs_all.at[srow]],
                                  ss[b]).wait()

        pltpu.async_copy(x_hbm.at[idxn_all.at[off + cc]], rows[b],
                         sg[b]).wait()
        pltpu.async_copy(rows[b], zsh.at[idxs_all.at[srow]], ss[b], add=True)

    def step1(t, carry):
        for b in range(2):
            cc = 2 * t + b
            chunk_body(b, cc, off + cc, cc >= 2)
        return carry

    lax.fori_loop(0, _PH1 // 2, step1, 0)

    for b in range(2):
        pltpu.make_async_copy(rows[b], zsh.at[idxs_all.at[off]], ss[b]).wait()
    pltpu.sync_copy(ti_hbm.at[pl.ds(A + _PH1, _SROWS)], idxs_all)

    def step2(t, carry):
        for b in range(2):
            cc = 2 * t + b

            @pl.when(cc < n)
            def _chunk(b=b, cc=cc):
                chunk_body(b, cc, off + cc - _PH1, cc >= _PH1 + 2)

        return carry

    nhalf = (n + 1) // 2
    lax.fori_loop(_PH1 // 2, nhalf, step2, 0)

    for b in range(2):
        pltpu.make_async_copy(rows[b], zsh.at[idxs_all.at[off]], ss[b]).wait()
    plsc.subcore_barrier()

    @pl.when(s < _NS - 1)
    def _out_main():
        pltpu.sync_copy(zsh.at[pl.ds(r0, _RPT)],
                        out_hbm.at[pl.ds(c * _NSEG + r0, _RPT)])

    @pl.when(s == _NS - 1)
    def _out_last():
        pltpu.sync_copy(zsh.at[pl.ds(r0, _RPT_LAST)],
                        out_hbm.at[pl.ds(c * _NSEG + r0, _RPT_LAST)])


_sc_segment_sum = functools.partial(
    pl.kernel,
    out_type=jax.ShapeDtypeStruct((_NC * _NSEG, _D), jnp.float32),
    mesh=plsc.VectorSubcoreMesh(core_axis_name="c", subcore_axis_name="s",
                                num_cores=_NC, num_subcores=_NS),
    scratch_types=[
        pltpu.VMEM((_IDXROWS, _K), jnp.int32),
        pltpu.VMEM((_SROWS, _K), jnp.int32),
        pltpu.VMEM((_K, _D), jnp.float32),
        pltpu.VMEM((_K, _D), jnp.float32),
        pltpu.VMEM_SHARED((_NSEG, _D), jnp.float32),
        pltpu.SemaphoreType.DMA,
        pltpu.SemaphoreType.DMA,
        pltpu.SemaphoreType.DMA,
        pltpu.SemaphoreType.DMA,
        pltpu.SemaphoreType.DMA,
        pltpu.SemaphoreType.DMA,
    ],
)(_sc_body)


def _mlp_body(zp_ref, w1_ref, b1_ref, w2t_ref, b2_ref, o_ref):
    z = zp_ref[0:_NSEG, :] + zp_ref[_NSEG:2 * _NSEG, :]
    h = jnp.dot(z, w1_ref[...], preferred_element_type=jnp.float32)
    h = jnp.maximum(h + b1_ref[...], 0.0)
    logit = jnp.sum(h * w2t_ref[...], axis=1, keepdims=True) + b2_ref[...]
    o_ref[...] = jax.nn.sigmoid(logit)


_mlp = pl.pallas_call(
    _mlp_body,
    out_shape=jax.ShapeDtypeStruct((_NSEG, 1), jnp.float32),
)


def kernel(x, target_nodes, target_ids, W1, b1, W2, b2):
    pad = _GPAD * _K - _N_EDGES
    tn = jnp.pad(target_nodes.astype(jnp.int32), (0, pad)).reshape(_GPAD, _K)
    ti = jnp.pad(target_ids.astype(jnp.int32), (0, pad)).reshape(_GPAD, _K)
    zeros = jnp.zeros((_NSEG - (_NS - 1) * _RPT, _D), jnp.float32)
    zparts = _sc_segment_sum(x, tn, ti, zeros)
    out = _mlp(zparts, W1, b1.reshape(1, _D), W2.reshape(1, _D),
               b2.reshape(1, 1))
    return out.reshape(_NSEG)
